# baseline (speedup 1.0000x reference)
import functools

import jax
import jax.numpy as jnp
from jax import lax
from jax.experimental import pallas as pl
from jax.experimental.pallas import tpu as pltpu
from jax.experimental.pallas import tpu_sc as plsc

D = 64
S = 200
NC = 2
NS = 16
NW = NC * NS
CHUNK = 128


def _body(idx_hbm, pos_hbm, table_hbm, out_hbm, idx_v, rows_v, pos_v, sem,
          *, rows_per_w, n_chunks):
    wid = lax.axis_index("s") * NC + lax.axis_index("c")
    base = wid * rows_per_w
    pltpu.sync_copy(pos_hbm, pos_v)

    def chunk_body(c, carry):
        start = base + c * CHUNK
        pltpu.sync_copy(idx_hbm.at[pl.ds(start, CHUNK)], idx_v)
        pltpu.async_copy(table_hbm.at[idx_v], rows_v, sem).wait()
        p0 = lax.rem(c * CHUNK, S)

        def add_body(r, carry2):
            pr = p0 + r
            for j in range(D // 16):
                sl = pl.ds(j * 16, 16)
                rows_v[r, sl] = rows_v[r, sl] + pos_v[pr, sl]
            return carry2

        lax.fori_loop(0, CHUNK, add_body, 0)
        pltpu.sync_copy(rows_v, out_hbm.at[pl.ds(start, CHUNK)])
        return carry

    lax.fori_loop(0, n_chunks, chunk_body, 0)


def kernel(output, word_table, pos_table):
    batch, seq = output.shape
    n = batch * seq
    assert n % (NW * CHUNK) == 0
    rows_per_w = n // NW
    n_chunks = rows_per_w // CHUNK

    idx_flat = output.reshape(n).astype(jnp.int32)
    pos_dup = jnp.concatenate([pos_table, pos_table[:CHUNK]], axis=0)

    mesh = plsc.VectorSubcoreMesh(core_axis_name="c", subcore_axis_name="s")
    k = functools.partial(
        pl.kernel,
        mesh=mesh,
        out_type=jax.ShapeDtypeStruct((n, D), jnp.float32),
        scratch_types=[
            pltpu.VMEM((CHUNK,), jnp.int32),
            pltpu.VMEM((CHUNK, D), jnp.float32),
            pltpu.VMEM((S + CHUNK, D), jnp.float32),
            pltpu.SemaphoreType.DMA,
        ],
        compiler_params=pltpu.CompilerParams(use_tc_tiling_on_sc=False),
    )(functools.partial(_body, rows_per_w=rows_per_w, n_chunks=n_chunks))

    out_flat = k(idx_flat, pos_dup, word_table)
    return out_flat.reshape(batch, seq, D)


# speedup vs baseline: 1.6008x; 1.6008x over previous
import functools

import jax
import jax.numpy as jnp
from jax import lax
from jax.experimental import pallas as pl
from jax.experimental.pallas import tpu as pltpu
from jax.experimental.pallas import tpu_sc as plsc

D = 64
S = 200
NC = 2
NS = 16
NW = NC * NS
CHUNK = 128
NBUF = 4


def _body(idx_hbm, pos_hbm, table_hbm, out_hbm, idx_v, pos_v,
          rows0, rows1, rows2, rows3,
          gsem0, gsem1, gsem2, gsem3, osem0, osem1, osem2, osem3,
          *, rows_per_w, n_chunks):
    rows = (rows0, rows1, rows2, rows3)
    gsem = (gsem0, gsem1, gsem2, gsem3)
    osem = (osem0, osem1, osem2, osem3)

    wid = lax.axis_index("s") * NC + lax.axis_index("c")
    base = wid * rows_per_w

    pltpu.sync_copy(idx_hbm.at[wid], idx_v)
    pltpu.sync_copy(pos_hbm, pos_v)

    def gather_start(c, b):
        pltpu.async_copy(table_hbm.at[idx_v.at[c]], rows[b], gsem[b])

    def gather_wait(c, b):
        pltpu.make_async_copy(table_hbm.at[idx_v.at[c]], rows[b],
                              gsem[b]).wait()

    def out_start(c, b):
        pltpu.async_copy(rows[b], out_hbm.at[pl.ds(base + c * CHUNK, CHUNK)],
                         osem[b])

    def out_wait(c, b):
        pltpu.make_async_copy(rows[b],
                              out_hbm.at[pl.ds(base + c * CHUNK, CHUNK)],
                              osem[b]).wait()

    def add_pos(c, b):
        p0 = lax.rem(c * CHUNK, S)
        rows_b = rows[b]

        @plsc.parallel_loop(0, CHUNK, unroll=4)
        def _(r):
            pr = p0 + r
            for j in range(D // 16):
                sl = pl.ds(j * 16, 16)
                rows_b[r, sl] = rows_b[r, sl] + pos_v[pr, sl]

    for b in range(NBUF - 1):
        gather_start(b, b)

    def ring_step(g, carry):
        for b in range(NBUF):
            c = g * NBUF + b
            gather_wait(c, b)
            add_pos(c, b)
            b_prev = (b - 1) % NBUF
            if b == 0:
                @pl.when(g > 0)
                def _():
                    out_wait(c - 1, b_prev)
            else:
                out_wait(c - 1, b_prev)
            out_start(c, b)

            nxt = c + NBUF - 1
            b_nxt = (b + NBUF - 1) % NBUF

            @pl.when(nxt < n_chunks)
            def _():
                gather_start(nxt, b_nxt)
        return carry

    lax.fori_loop(0, n_chunks // NBUF, ring_step, 0)
    out_wait(n_chunks - 1, (n_chunks - 1) % NBUF)


def kernel(output, word_table, pos_table):
    batch, seq = output.shape
    n = batch * seq
    assert n % (NW * CHUNK) == 0
    rows_per_w = n // NW
    n_chunks = rows_per_w // CHUNK
    assert n_chunks % NBUF == 0

    idx = output.reshape(NW, n_chunks, CHUNK).astype(jnp.int32)
    pos_dup = jnp.concatenate([pos_table, pos_table[:CHUNK]], axis=0)

    mesh = plsc.VectorSubcoreMesh(core_axis_name="c", subcore_axis_name="s")
    k = functools.partial(
        pl.kernel,
        mesh=mesh,
        out_type=jax.ShapeDtypeStruct((n, D), jnp.float32),
        scratch_types=[
            pltpu.VMEM((n_chunks, CHUNK), jnp.int32),
            pltpu.VMEM((S + CHUNK, D), jnp.float32),
        ] + [pltpu.VMEM((CHUNK, D), jnp.float32)] * NBUF
          + [pltpu.SemaphoreType.DMA] * (2 * NBUF),
        compiler_params=pltpu.CompilerParams(use_tc_tiling_on_sc=False),
    )(functools.partial(_body, rows_per_w=rows_per_w, n_chunks=n_chunks))

    out_flat = k(idx, pos_dup, word_table)
    return out_flat.reshape(batch, seq, D)
